# Initial kernel scaffold; baseline (speedup 1.0000x reference)
import jax
import jax.numpy as jnp
from jax.experimental import pallas as pl


def kernel(x, vocab):
    raise NotImplementedError("write your pallas kernel here")


# speedup vs baseline: 3.1036x; 3.1036x over previous
import jax
import jax.numpy as jnp
from jax.experimental import pallas as pl
from jax.experimental.pallas import tpu as pltpu
from jax.experimental.pallas import tpu_sc as plsc

_WINDOW = 128


def kernel(x, vocab):
    B, S = x.shape
    V, D = vocab.shape
    N = B * S
    assert N % _WINDOW == 0

    idx = x.reshape(1, N).astype(jnp.int32)
    mesh = plsc.VectorSubcoreMesh(core_axis_name="core", subcore_axis_name="subcore")

    @pl.kernel(
        out_type=jax.ShapeDtypeStruct((N, D), vocab.dtype),
        mesh=mesh,
    )
    def gather_kernel(vocab_hbm, idx_hbm, out_hbm):
        def body(i_vmem, o_vmem):
            pltpu.sync_copy(vocab_hbm.at[i_vmem.at[0]], o_vmem)

        pltpu.emit_pipeline(
            body,
            grid=(N // _WINDOW,),
            in_specs=[pl.BlockSpec((1, _WINDOW), index_map=lambda i: (0, i))],
            out_specs=[pl.BlockSpec((_WINDOW, D), index_map=lambda i: (i, 0))],
            core_axis_name=("core", "subcore"),
            dimension_semantics=(pltpu.PARALLEL,),
        )(idx_hbm, out_hbm)

    return gather_kernel(vocab, idx).reshape(B, S, D)


# speedup vs baseline: 4.2218x; 1.3603x over previous
import jax
import jax.numpy as jnp
from jax.experimental import pallas as pl
from jax.experimental.pallas import tpu as pltpu
from jax.experimental.pallas import tpu_sc as plsc

_K = 8


def kernel(x, vocab):
    B, S = x.shape
    V, D = vocab.shape
    assert B % _K == 0

    idx = x.astype(jnp.int32)
    mesh = plsc.VectorSubcoreMesh(core_axis_name="core", subcore_axis_name="subcore")

    @pl.kernel(
        out_type=jax.ShapeDtypeStruct((B, S, D), vocab.dtype),
        mesh=mesh,
        compiler_params=pltpu.CompilerParams(use_tc_tiling_on_sc=True),
    )
    def gather_kernel(vocab_hbm, idx_hbm, out_hbm):
        def body(i_vmem, o_vmem):
            for k in range(_K):
                pltpu.sync_copy(vocab_hbm.at[i_vmem.at[k]], o_vmem.at[k])

        pltpu.emit_pipeline(
            body,
            grid=(B // _K,),
            in_specs=[pl.BlockSpec((_K, S), index_map=lambda i: (i, 0))],
            out_specs=[pl.BlockSpec((_K, S, D), index_map=lambda i: (i, 0, 0))],
            core_axis_name=("core", "subcore"),
            dimension_semantics=(pltpu.PARALLEL,),
        )(idx_hbm, out_hbm)

    return gather_kernel(vocab, idx)


# speedup vs baseline: 8.5135x; 2.0166x over previous
import jax
import jax.numpy as jnp
from jax.experimental import pallas as pl
from jax.experimental.pallas import tpu as pltpu
from jax.experimental.pallas import tpu_sc as plsc

_WINDOW = 128


def kernel(x, vocab):
    B, S = x.shape
    V, D = vocab.shape
    N = B * S
    assert N % _WINDOW == 0

    idx = jnp.transpose(x).reshape(1, N).astype(jnp.int32)
    mesh = plsc.VectorSubcoreMesh(core_axis_name="core", subcore_axis_name="subcore")

    @pl.kernel(
        out_type=jax.ShapeDtypeStruct((N, D), vocab.dtype),
        mesh=mesh,
    )
    def gather_kernel(vocab_hbm, idx_hbm, out_hbm):
        def body(i_vmem, o_vmem):
            pltpu.sync_copy(vocab_hbm.at[i_vmem.at[0]], o_vmem)

        pltpu.emit_pipeline(
            body,
            grid=(N // _WINDOW,),
            in_specs=[pl.BlockSpec((1, _WINDOW), index_map=lambda i: (0, i))],
            out_specs=[pl.BlockSpec((_WINDOW, D), index_map=lambda i: (i, 0))],
            core_axis_name=("core", "subcore"),
            dimension_semantics=(pltpu.PARALLEL,),
        )(idx_hbm, out_hbm)

    out_sb = gather_kernel(vocab, idx).reshape(S, B, D)
    return jnp.transpose(out_sb, (1, 0, 2))


# speedup vs baseline: 10.4350x; 1.2257x over previous
import jax
import jax.numpy as jnp
from jax.experimental import pallas as pl
from jax.experimental.pallas import tpu as pltpu
from jax.experimental.pallas import tpu_sc as plsc

_WINDOW = 128
_K = 2


def kernel(x, vocab):
    B, S = x.shape
    V, D = vocab.shape
    N = B * S
    assert N % (_WINDOW * _K) == 0

    idx = jnp.transpose(x).reshape(N // _WINDOW, _WINDOW).astype(jnp.int32)
    mesh = plsc.VectorSubcoreMesh(core_axis_name="core", subcore_axis_name="subcore")

    @pl.kernel(
        out_type=jax.ShapeDtypeStruct((N, D), vocab.dtype),
        mesh=mesh,
        scratch_types=[pltpu.SemaphoreType.DMA, pltpu.SemaphoreType.DMA],
    )
    def gather_kernel(vocab_hbm, idx_hbm, out_hbm, sem0, sem1):
        def body(i_vmem, o_vmem):
            c0 = pltpu.async_copy(
                vocab_hbm.at[i_vmem.at[0]], o_vmem.at[pl.ds(0, _WINDOW)], sem0
            )
            c1 = pltpu.async_copy(
                vocab_hbm.at[i_vmem.at[1]], o_vmem.at[pl.ds(_WINDOW, _WINDOW)], sem1
            )
            c0.wait()
            c1.wait()

        pltpu.emit_pipeline(
            body,
            grid=(N // (_WINDOW * _K),),
            in_specs=[pl.BlockSpec((_K, _WINDOW), index_map=lambda i: (i, 0))],
            out_specs=[pl.BlockSpec((_K * _WINDOW, D), index_map=lambda i: (i, 0))],
            core_axis_name=("core", "subcore"),
            dimension_semantics=(pltpu.PARALLEL,),
        )(idx_hbm, out_hbm)

    out_sb = gather_kernel(vocab, idx).reshape(S, B, D)
    return jnp.transpose(out_sb, (1, 0, 2))
